# Initial kernel scaffold; baseline (speedup 1.0000x reference)
import jax
import jax.numpy as jnp
from jax.experimental import pallas as pl


def kernel(features, xyz, fc1_w, fc1_b, fc2_w, fc2_b, fd1_w, fd1_b, fd2_w, fd2_b, wq, wk, wv):
    raise NotImplementedError("write your pallas kernel here")


# speedup vs baseline: 7.8474x; 7.8474x over previous
import jax
import jax.numpy as jnp
from jax.experimental import pallas as pl
from jax.experimental.pallas import tpu as pltpu

K_SP = 16
K_TMP = 8
TR = 256
BIG = 1e30

_DIMS_T = (((1,), (1,)), ((), ()))


def _bf(x):
    return x.astype(jnp.bfloat16)


def _f32(x):
    return x.astype(jnp.float32)


def _bdot(a, b):
    return jnp.dot(_bf(a), _bf(b), preferred_element_type=jnp.float32)


def _bdot_t(a, b):
    return jax.lax.dot_general(_bf(a), _bf(b), _DIMS_T,
                               preferred_element_type=jnp.float32)


def _argmin_oh(rank, lanes, n):
    m = jnp.min(rank, axis=1, keepdims=True)
    am = jnp.min(jnp.where(rank == m, lanes, n), axis=1, keepdims=True)
    return lanes == am


def _spatial_kernel(feat_ref, xt_ref, xf_ref, sqt_ref,
                    fc1w_ref, fc1b_ref, wq_ref, wk_ref, wv_ref,
                    fd1w_ref, fd1b_ref, fd2w_ref, fd2b_ref,
                    out_ref, x_s, xk_hi, xk_lo, xv_hi, xv_lo, xf_hi, xf_lo):
    r = pl.program_id(1)
    n = xf_ref.shape[1]

    @pl.when(r == 0)
    def _():
        x = _bdot(feat_ref[0], fc1w_ref[...]) + fc1b_ref[...]
        x_s[...] = x
        xk = _bdot(x, wk_ref[...])
        kh = _bf(xk)
        xk_hi[...] = kh
        xk_lo[...] = _bf(xk - _f32(kh))
        xv = _bdot(x, wv_ref[...])
        vh = _bf(xv)
        xv_hi[...] = vh
        xv_lo[...] = _bf(xv - _f32(vh))
        xf = xf_ref[0]
        fh = _bf(xf)
        xf_hi[...] = fh
        xf_lo[...] = _bf(xf - _f32(fh))

    base = r * TR
    xt = x_s[pl.ds(base, TR), :]
    qf = _bdot(xt, wq_ref[...])

    xi = xt_ref[0]
    dot = _bdot_t(xi, xf_ref[0])
    sq_i = jnp.sum(xi * xi, axis=1, keepdims=True)
    rank = (sq_i + sqt_ref[0]) - 2.0 * dot

    lanes = jax.lax.broadcasted_iota(jnp.int32, (TR, n), 1)

    logits = []
    vps = []
    for _ in range(K_SP):
        h = _argmin_oh(rank, lanes, n)
        rank = jnp.where(h, BIG, rank)
        hb = h.astype(jnp.bfloat16)
        xj = (jnp.dot(hb, xf_hi[...], preferred_element_type=jnp.float32)
              + jnp.dot(hb, xf_lo[...], preferred_element_type=jnp.float32))
        kj = (jnp.dot(hb, xk_hi[...], preferred_element_type=jnp.float32)
              + jnp.dot(hb, xk_lo[...], preferred_element_type=jnp.float32))
        vj = (jnp.dot(hb, xv_hi[...], preferred_element_type=jnp.float32)
              + jnp.dot(hb, xv_lo[...], preferred_element_type=jnp.float32))
        delta = xi - xj
        p1 = jnp.maximum(_bdot(delta, fd1w_ref[...]) + fd1b_ref[...], 0.0)
        pos = _bdot(p1, fd2w_ref[...]) + fd2b_ref[...]
        logits.append(jnp.sum(qf * (kj + pos), axis=1, keepdims=True) * 0.125)
        vps.append(vj + pos)

    lg = jnp.concatenate(logits, axis=1)
    lg = lg - jnp.max(lg, axis=1, keepdims=True)
    e = jnp.exp(lg)
    a = e / jnp.sum(e, axis=1, keepdims=True)

    acc = jnp.zeros((TR, out_ref.shape[2]), jnp.float32)
    for k in range(K_SP):
        acc = acc + a[:, k:k + 1] * vps[k]
    out_ref[0] = acc


def _temporal_kernel(sp_ref, fc2a_ref, fc2b_ref, fc2bias_ref,
                     out_ref, fn_s, sp_hi, sp_lo):
    r = pl.program_id(1)
    n = sp_ref.shape[1]

    @pl.when(r == 0)
    def _():
        sp = sp_ref[0]
        ssq = jnp.sum(sp * sp, axis=1, keepdims=True)
        nrm = jnp.maximum(jnp.sqrt(ssq), 1e-12)
        fn_s[...] = sp / nrm
        sh = _bf(sp)
        sp_hi[...] = sh
        sp_lo[...] = _bf(sp - _f32(sh))

    base = r * TR
    spt = sp_ref[0, pl.ds(base, TR), :]
    fnt = fn_s[pl.ds(base, TR), :]

    rank = -_bdot_t(fnt, fn_s[...])

    lanes = jax.lax.broadcasted_iota(jnp.int32, (TR, n), 1)

    logits = []
    kts = []
    for _ in range(K_TMP):
        h = _argmin_oh(rank, lanes, n)
        rank = jnp.where(h, BIG, rank)
        hb = h.astype(jnp.bfloat16)
        ktj = (jnp.dot(hb, sp_hi[...], preferred_element_type=jnp.float32)
               + jnp.dot(hb, sp_lo[...], preferred_element_type=jnp.float32))
        logits.append(jnp.sum(spt * ktj, axis=1, keepdims=True) * 0.125)
        kts.append(ktj)

    lg = jnp.concatenate(logits, axis=1)
    lg = lg - jnp.max(lg, axis=1, keepdims=True)
    e = jnp.exp(lg)
    a = e / jnp.sum(e, axis=1, keepdims=True)

    temporal = jnp.zeros((TR, sp_ref.shape[2]), jnp.float32)
    for k in range(K_TMP):
        temporal = temporal + a[:, k:k + 1] * kts[k]

    out_ref[0] = (_bdot(spt, fc2a_ref[...]) + _bdot(temporal, fc2b_ref[...])
                  + fc2bias_ref[...])


@jax.jit
def kernel(features, xyz, fc1_w, fc1_b, fc2_w, fc2_b,
           fd1_w, fd1_b, fd2_w, fd2_b, wq, wk, wv):
    b, t, n, dp = features.shape
    f = b * t
    dm = fc1_w.shape[1]
    nt = n // TR

    feat = features.reshape(f, n, dp)
    pts = xyz.reshape(f, n, 3)
    xyzp = jnp.concatenate([pts, jnp.zeros((f, n, 5), jnp.float32)],
                           axis=-1)
    sqt = jnp.sum(pts * pts, axis=-1)[:, None, :]
    fd1w_p = jnp.concatenate([fd1_w, jnp.zeros((5, dm), jnp.float32)], axis=0)

    fc1b2 = fc1_b.reshape(1, dm)
    fd1b2 = fd1_b.reshape(1, dm)
    fd2b2 = fd2_b.reshape(1, dm)
    fc2b2 = fc2_b.reshape(1, dp)

    frame_spec = lambda shp: pl.BlockSpec(shp, lambda i, j: (i, 0, 0))
    tile_spec = lambda shp: pl.BlockSpec(shp, lambda i, j: (i, j, 0))
    w_spec = lambda shp: pl.BlockSpec(shp, lambda i, j: (0,) * len(shp))

    spatial = pl.pallas_call(
        _spatial_kernel,
        grid=(f, nt),
        in_specs=[
            frame_spec((1, n, dp)),
            tile_spec((1, TR, 8)),
            frame_spec((1, n, 8)),
            pl.BlockSpec((1, 1, n), lambda i, j: (i, 0, 0)),
            w_spec((dp, dm)), w_spec((1, dm)),
            w_spec((dm, dm)), w_spec((dm, dm)), w_spec((dm, dm)),
            w_spec((8, dm)), w_spec((1, dm)),
            w_spec((dm, dm)), w_spec((1, dm)),
        ],
        out_specs=tile_spec((1, TR, dm)),
        out_shape=jax.ShapeDtypeStruct((f, n, dm), jnp.float32),
        scratch_shapes=[pltpu.VMEM((n, dm), jnp.float32),
                        pltpu.VMEM((n, dm), jnp.bfloat16),
                        pltpu.VMEM((n, dm), jnp.bfloat16),
                        pltpu.VMEM((n, dm), jnp.bfloat16),
                        pltpu.VMEM((n, dm), jnp.bfloat16),
                        pltpu.VMEM((n, 8), jnp.bfloat16),
                        pltpu.VMEM((n, 8), jnp.bfloat16)],
    )(feat, xyzp, xyzp, sqt, fc1_w, fc1b2, wq, wk, wv,
      fd1w_p, fd1b2, fd2_w, fd2b2)

    out = pl.pallas_call(
        _temporal_kernel,
        grid=(f, nt),
        in_specs=[
            frame_spec((1, n, dm)),
            w_spec((dm, dp)), w_spec((dm, dp)), w_spec((1, dp)),
        ],
        out_specs=tile_spec((1, TR, dp)),
        out_shape=jax.ShapeDtypeStruct((f, n, dp), jnp.float32),
        scratch_shapes=[pltpu.VMEM((n, dm), jnp.float32),
                        pltpu.VMEM((n, dm), jnp.bfloat16),
                        pltpu.VMEM((n, dm), jnp.bfloat16)],
    )(spatial, fc2_w[:dm], fc2_w[dm:], fc2b2)

    return jnp.transpose(out.reshape(b, t, n, dp), (0, 1, 3, 2))


# speedup vs baseline: 10.2335x; 1.3041x over previous
import jax
import jax.numpy as jnp
from jax.experimental import pallas as pl
from jax.experimental.pallas import tpu as pltpu

K_SP = 16
K_TMP = 8
TR = 256
BIG = 1e30

_DIMS_T = (((1,), (1,)), ((), ()))


def _bf(x):
    return x.astype(jnp.bfloat16)


def _f32(x):
    return x.astype(jnp.float32)


def _bdot(a, b):
    return jnp.dot(_bf(a), _bf(b), preferred_element_type=jnp.float32)


def _bdot_t(a, b):
    return jax.lax.dot_general(_bf(a), _bf(b), _DIMS_T,
                               preferred_element_type=jnp.float32)


def _argmin_oh(rank, lanes, n):
    am = jnp.argmin(rank, axis=1).astype(jnp.int32)[:, None]
    return lanes == am


def _spatial_kernel(feat_ref, xt_ref, xf_ref, sqt_ref,
                    fc1w_ref, fc1b_ref, wq_ref, wk_ref, wv_ref,
                    fd1w_ref, fd1b_ref, fd2w_ref, fd2b_ref,
                    out_ref, x_s, xk_hi, xk_lo, xv_hi, xv_lo, xf_hi, xf_lo):
    r = pl.program_id(1)
    n = xf_ref.shape[1]

    @pl.when(r == 0)
    def _():
        x = _bdot(feat_ref[0], fc1w_ref[...]) + fc1b_ref[...]
        x_s[...] = x
        xk = _bdot(x, wk_ref[...])
        kh = _bf(xk)
        xk_hi[...] = kh
        xk_lo[...] = _bf(xk - _f32(kh))
        xv = _bdot(x, wv_ref[...])
        vh = _bf(xv)
        xv_hi[...] = vh
        xv_lo[...] = _bf(xv - _f32(vh))
        xf = xf_ref[0]
        fh = _bf(xf)
        xf_hi[...] = fh
        xf_lo[...] = _bf(xf - _f32(fh))

    base = r * TR
    xt = x_s[pl.ds(base, TR), :]
    qf = _bdot(xt, wq_ref[...])

    xi = xt_ref[0]
    dot = _bdot_t(xi, xf_ref[0])
    sq_i = jnp.sum(xi * xi, axis=1, keepdims=True)
    rank = (sq_i + sqt_ref[0]) - 2.0 * dot

    lanes = jax.lax.broadcasted_iota(jnp.int32, (TR, n), 1)

    logits = []
    vps = []
    for _ in range(K_SP):
        h = _argmin_oh(rank, lanes, n)
        rank = jnp.where(h, BIG, rank)
        hb = h.astype(jnp.bfloat16)
        xj = (jnp.dot(hb, xf_hi[...], preferred_element_type=jnp.float32)
              + jnp.dot(hb, xf_lo[...], preferred_element_type=jnp.float32))
        kj = (jnp.dot(hb, xk_hi[...], preferred_element_type=jnp.float32)
              + jnp.dot(hb, xk_lo[...], preferred_element_type=jnp.float32))
        vj = (jnp.dot(hb, xv_hi[...], preferred_element_type=jnp.float32)
              + jnp.dot(hb, xv_lo[...], preferred_element_type=jnp.float32))
        delta = xi - xj
        p1 = jnp.maximum(_bdot(delta, fd1w_ref[...]) + fd1b_ref[...], 0.0)
        pos = _bdot(p1, fd2w_ref[...]) + fd2b_ref[...]
        logits.append(jnp.sum(qf * (kj + pos), axis=1, keepdims=True) * 0.125)
        vps.append(vj + pos)

    lg = jnp.concatenate(logits, axis=1)
    lg = lg - jnp.max(lg, axis=1, keepdims=True)
    e = jnp.exp(lg)
    a = e / jnp.sum(e, axis=1, keepdims=True)

    acc = jnp.zeros((TR, out_ref.shape[2]), jnp.float32)
    for k in range(K_SP):
        acc = acc + a[:, k:k + 1] * vps[k]
    out_ref[0] = acc


def _temporal_kernel(sp_ref, fc2a_ref, fc2b_ref, fc2bias_ref,
                     out_ref, fn_s, sp_hi, sp_lo):
    r = pl.program_id(1)
    n = sp_ref.shape[1]

    @pl.when(r == 0)
    def _():
        sp = sp_ref[0]
        ssq = jnp.sum(sp * sp, axis=1, keepdims=True)
        nrm = jnp.maximum(jnp.sqrt(ssq), 1e-12)
        fn_s[...] = sp / nrm
        sh = _bf(sp)
        sp_hi[...] = sh
        sp_lo[...] = _bf(sp - _f32(sh))

    base = r * TR
    spt = sp_ref[0, pl.ds(base, TR), :]
    fnt = fn_s[pl.ds(base, TR), :]

    rank = -_bdot_t(fnt, fn_s[...])

    lanes = jax.lax.broadcasted_iota(jnp.int32, (TR, n), 1)

    logits = []
    kts = []
    for _ in range(K_TMP):
        h = _argmin_oh(rank, lanes, n)
        rank = jnp.where(h, BIG, rank)
        hb = h.astype(jnp.bfloat16)
        ktj = (jnp.dot(hb, sp_hi[...], preferred_element_type=jnp.float32)
               + jnp.dot(hb, sp_lo[...], preferred_element_type=jnp.float32))
        logits.append(jnp.sum(spt * ktj, axis=1, keepdims=True) * 0.125)
        kts.append(ktj)

    lg = jnp.concatenate(logits, axis=1)
    lg = lg - jnp.max(lg, axis=1, keepdims=True)
    e = jnp.exp(lg)
    a = e / jnp.sum(e, axis=1, keepdims=True)

    temporal = jnp.zeros((TR, sp_ref.shape[2]), jnp.float32)
    for k in range(K_TMP):
        temporal = temporal + a[:, k:k + 1] * kts[k]

    out_ref[0] = (_bdot(spt, fc2a_ref[...]) + _bdot(temporal, fc2b_ref[...])
                  + fc2bias_ref[...])


@jax.jit
def kernel(features, xyz, fc1_w, fc1_b, fc2_w, fc2_b,
           fd1_w, fd1_b, fd2_w, fd2_b, wq, wk, wv):
    b, t, n, dp = features.shape
    f = b * t
    dm = fc1_w.shape[1]
    nt = n // TR

    feat = features.reshape(f, n, dp)
    pts = xyz.reshape(f, n, 3)
    xyzp = jnp.concatenate([pts, jnp.zeros((f, n, 5), jnp.float32)],
                           axis=-1)
    sqt = jnp.sum(pts * pts, axis=-1)[:, None, :]
    fd1w_p = jnp.concatenate([fd1_w, jnp.zeros((5, dm), jnp.float32)], axis=0)

    fc1b2 = fc1_b.reshape(1, dm)
    fd1b2 = fd1_b.reshape(1, dm)
    fd2b2 = fd2_b.reshape(1, dm)
    fc2b2 = fc2_b.reshape(1, dp)

    frame_spec = lambda shp: pl.BlockSpec(shp, lambda i, j: (i, 0, 0))
    tile_spec = lambda shp: pl.BlockSpec(shp, lambda i, j: (i, j, 0))
    w_spec = lambda shp: pl.BlockSpec(shp, lambda i, j: (0,) * len(shp))

    spatial = pl.pallas_call(
        _spatial_kernel,
        grid=(f, nt),
        in_specs=[
            frame_spec((1, n, dp)),
            tile_spec((1, TR, 8)),
            frame_spec((1, n, 8)),
            pl.BlockSpec((1, 1, n), lambda i, j: (i, 0, 0)),
            w_spec((dp, dm)), w_spec((1, dm)),
            w_spec((dm, dm)), w_spec((dm, dm)), w_spec((dm, dm)),
            w_spec((8, dm)), w_spec((1, dm)),
            w_spec((dm, dm)), w_spec((1, dm)),
        ],
        out_specs=tile_spec((1, TR, dm)),
        out_shape=jax.ShapeDtypeStruct((f, n, dm), jnp.float32),
        scratch_shapes=[pltpu.VMEM((n, dm), jnp.float32),
                        pltpu.VMEM((n, dm), jnp.bfloat16),
                        pltpu.VMEM((n, dm), jnp.bfloat16),
                        pltpu.VMEM((n, dm), jnp.bfloat16),
                        pltpu.VMEM((n, dm), jnp.bfloat16),
                        pltpu.VMEM((n, 8), jnp.bfloat16),
                        pltpu.VMEM((n, 8), jnp.bfloat16)],
    )(feat, xyzp, xyzp, sqt, fc1_w, fc1b2, wq, wk, wv,
      fd1w_p, fd1b2, fd2_w, fd2b2)

    out = pl.pallas_call(
        _temporal_kernel,
        grid=(f, nt),
        in_specs=[
            frame_spec((1, n, dm)),
            w_spec((dm, dp)), w_spec((dm, dp)), w_spec((1, dp)),
        ],
        out_specs=tile_spec((1, TR, dp)),
        out_shape=jax.ShapeDtypeStruct((f, n, dp), jnp.float32),
        scratch_shapes=[pltpu.VMEM((n, dm), jnp.float32),
                        pltpu.VMEM((n, dm), jnp.bfloat16),
                        pltpu.VMEM((n, dm), jnp.bfloat16)],
    )(spatial, fc2_w[:dm], fc2_w[dm:], fc2b2)

    return jnp.transpose(out.reshape(b, t, n, dp), (0, 1, 3, 2))


# speedup vs baseline: 13.9265x; 1.3609x over previous
import jax
import jax.numpy as jnp
from jax.experimental import pallas as pl
from jax.experimental.pallas import tpu as pltpu

K_SP = 16
K_TMP = 8
TR = 256
BIG = 1e30

_DIMS_T = (((1,), (1,)), ((), ()))


def _bf(x):
    return x.astype(jnp.bfloat16)


def _f32(x):
    return x.astype(jnp.float32)


def _bdot(a, b):
    return jnp.dot(_bf(a), _bf(b), preferred_element_type=jnp.float32)


def _bdot_t(a, b):
    return jax.lax.dot_general(_bf(a), _bf(b), _DIMS_T,
                               preferred_element_type=jnp.float32)


def _argmin_oh(rank, lanes, n):
    am = jnp.argmin(rank, axis=1).astype(jnp.int32)[:, None]
    return lanes == am


def _spatial_kernel(feat_ref, xt_ref, xf_ref, sqt_ref,
                    fc1w_ref, fc1b_ref, wq_ref, wk_ref, wv_ref,
                    fd1w_ref, fd1b_ref, fd2w_ref, fd2b_ref,
                    out_ref, x_s, cat_hi, cat_lo):
    r = pl.program_id(1)
    n = xf_ref.shape[1]

    @pl.when(r == 0)
    def _():
        x = _bdot(feat_ref[0], fc1w_ref[...]) + fc1b_ref[...]
        x_s[...] = x
        xk = _bdot(x, wk_ref[...])
        xv = _bdot(x, wv_ref[...])
        xf = xf_ref[0]
        kh = _bf(xk)
        vh = _bf(xv)
        fh = _bf(xf)
        cat_hi[...] = jnp.concatenate([kh, vh, fh], axis=1)
        cat_lo[...] = jnp.concatenate([_bf(xk - _f32(kh)),
                                       _bf(xv - _f32(vh)),
                                       _bf(xf - _f32(fh))], axis=1)

    base = r * TR
    xt = x_s[pl.ds(base, TR), :]
    qf = _bdot(xt, wq_ref[...])

    xi = xt_ref[0]
    dot = _bdot_t(xi, xf_ref[0])
    sq_i = jnp.sum(xi * xi, axis=1, keepdims=True)
    rank = (sq_i + sqt_ref[0]) - 2.0 * dot

    lanes = jax.lax.broadcasted_iota(jnp.int32, (TR, n), 1)

    hbs = []
    for _ in range(K_SP):
        h = _argmin_oh(rank, lanes, n)
        rank = jnp.where(h, BIG, rank)
        hbs.append(h.astype(jnp.bfloat16))

    hb_all = jnp.concatenate(hbs, axis=0)
    ej = (jnp.dot(hb_all, cat_hi[...], preferred_element_type=jnp.float32)
          + jnp.dot(hb_all, cat_lo[...], preferred_element_type=jnp.float32))
    dm = fd2w_ref.shape[1]
    kj_all = ej[:, :dm]
    vj_all = ej[:, dm:2 * dm]
    xj_all = ej[:, 2 * dm:]

    xi_rep = jnp.concatenate([xi] * K_SP, axis=0)
    delta = xi_rep - xj_all
    p1 = jnp.maximum(_bdot(delta, fd1w_ref[...]) + fd1b_ref[...], 0.0)
    pos = _bdot(p1, fd2w_ref[...]) + fd2b_ref[...]
    qf_rep = jnp.concatenate([qf] * K_SP, axis=0)
    lg_all = jnp.sum(qf_rep * (kj_all + pos), axis=1, keepdims=True) * 0.125
    vps = vj_all + pos

    lg = jnp.concatenate([lg_all[k * TR:(k + 1) * TR] for k in range(K_SP)],
                         axis=1)
    lg = lg - jnp.max(lg, axis=1, keepdims=True)
    e = jnp.exp(lg)
    a = e / jnp.sum(e, axis=1, keepdims=True)

    acc = jnp.zeros((TR, out_ref.shape[2]), jnp.float32)
    for k in range(K_SP):
        acc = acc + a[:, k:k + 1] * vps[k * TR:(k + 1) * TR]
    out_ref[0] = acc


def _temporal_kernel(sp_ref, fc2a_ref, fc2b_ref, fc2bias_ref,
                     out_ref, fn_s, sp_hi, sp_lo):
    r = pl.program_id(1)
    n = sp_ref.shape[1]

    @pl.when(r == 0)
    def _():
        sp = sp_ref[0]
        ssq = jnp.sum(sp * sp, axis=1, keepdims=True)
        nrm = jnp.maximum(jnp.sqrt(ssq), 1e-12)
        fn_s[...] = sp / nrm
        sh = _bf(sp)
        sp_hi[...] = sh
        sp_lo[...] = _bf(sp - _f32(sh))

    base = r * TR
    spt = sp_ref[0, pl.ds(base, TR), :]
    fnt = fn_s[pl.ds(base, TR), :]

    rank = -_bdot_t(fnt, fn_s[...])

    lanes = jax.lax.broadcasted_iota(jnp.int32, (TR, n), 1)

    hbs = []
    for _ in range(K_TMP):
        h = _argmin_oh(rank, lanes, n)
        rank = jnp.where(h, BIG, rank)
        hbs.append(h.astype(jnp.bfloat16))

    hb_all = jnp.concatenate(hbs, axis=0)
    kt_all = (jnp.dot(hb_all, sp_hi[...], preferred_element_type=jnp.float32)
              + jnp.dot(hb_all, sp_lo[...], preferred_element_type=jnp.float32))
    spt_rep = jnp.concatenate([spt] * K_TMP, axis=0)
    lg_all = jnp.sum(spt_rep * kt_all, axis=1, keepdims=True) * 0.125

    lg = jnp.concatenate([lg_all[k * TR:(k + 1) * TR] for k in range(K_TMP)],
                         axis=1)
    lg = lg - jnp.max(lg, axis=1, keepdims=True)
    e = jnp.exp(lg)
    a = e / jnp.sum(e, axis=1, keepdims=True)

    temporal = jnp.zeros((TR, sp_ref.shape[2]), jnp.float32)
    for k in range(K_TMP):
        temporal = temporal + a[:, k:k + 1] * kt_all[k * TR:(k + 1) * TR]

    out_ref[0] = (_bdot(spt, fc2a_ref[...]) + _bdot(temporal, fc2b_ref[...])
                  + fc2bias_ref[...])


@jax.jit
def kernel(features, xyz, fc1_w, fc1_b, fc2_w, fc2_b,
           fd1_w, fd1_b, fd2_w, fd2_b, wq, wk, wv):
    b, t, n, dp = features.shape
    f = b * t
    dm = fc1_w.shape[1]
    nt = n // TR

    feat = features.reshape(f, n, dp)
    pts = xyz.reshape(f, n, 3)
    xyzp = jnp.concatenate([pts, jnp.zeros((f, n, 5), jnp.float32)],
                           axis=-1)
    sqt = jnp.sum(pts * pts, axis=-1)[:, None, :]
    fd1w_p = jnp.concatenate([fd1_w, jnp.zeros((5, dm), jnp.float32)], axis=0)

    fc1b2 = fc1_b.reshape(1, dm)
    fd1b2 = fd1_b.reshape(1, dm)
    fd2b2 = fd2_b.reshape(1, dm)
    fc2b2 = fc2_b.reshape(1, dp)

    frame_spec = lambda shp: pl.BlockSpec(shp, lambda i, j: (i, 0, 0))
    tile_spec = lambda shp: pl.BlockSpec(shp, lambda i, j: (i, j, 0))
    w_spec = lambda shp: pl.BlockSpec(shp, lambda i, j: (0,) * len(shp))

    spatial = pl.pallas_call(
        _spatial_kernel,
        grid=(f, nt),
        in_specs=[
            frame_spec((1, n, dp)),
            tile_spec((1, TR, 8)),
            frame_spec((1, n, 8)),
            pl.BlockSpec((1, 1, n), lambda i, j: (i, 0, 0)),
            w_spec((dp, dm)), w_spec((1, dm)),
            w_spec((dm, dm)), w_spec((dm, dm)), w_spec((dm, dm)),
            w_spec((8, dm)), w_spec((1, dm)),
            w_spec((dm, dm)), w_spec((1, dm)),
        ],
        out_specs=tile_spec((1, TR, dm)),
        out_shape=jax.ShapeDtypeStruct((f, n, dm), jnp.float32),
        scratch_shapes=[pltpu.VMEM((n, dm), jnp.float32),
                        pltpu.VMEM((n, 2 * dm + 8), jnp.bfloat16),
                        pltpu.VMEM((n, 2 * dm + 8), jnp.bfloat16)],
    )(feat, xyzp, xyzp, sqt, fc1_w, fc1b2, wq, wk, wv,
      fd1w_p, fd1b2, fd2_w, fd2b2)

    out = pl.pallas_call(
        _temporal_kernel,
        grid=(f, nt),
        in_specs=[
            frame_spec((1, n, dm)),
            w_spec((dm, dp)), w_spec((dm, dp)), w_spec((1, dp)),
        ],
        out_specs=tile_spec((1, TR, dp)),
        out_shape=jax.ShapeDtypeStruct((f, n, dp), jnp.float32),
        scratch_shapes=[pltpu.VMEM((n, dm), jnp.float32),
                        pltpu.VMEM((n, dm), jnp.bfloat16),
                        pltpu.VMEM((n, dm), jnp.bfloat16)],
    )(spatial, fc2_w[:dm], fc2_w[dm:], fc2b2)

    return jnp.transpose(out.reshape(b, t, n, dp), (0, 1, 3, 2))
